# baseline (speedup 1.0000x reference)
import jax
import jax.numpy as jnp
from jax.experimental import pallas as pl
from jax.experimental.pallas import tpu as pltpu

NUM_EXPERTS = 8
NUM_GATES = 4
B, H, W, C = 4, 32, 32, 128
PIX = H * W
EALL = NUM_EXPERTS * C
K9 = 9 * C
F32 = jnp.float32
BF16 = jnp.bfloat16


def _shift_rows(v, s):
    if s > 0:
        return jnp.concatenate([v[s:], jnp.zeros((s, C), v.dtype)], axis=0)
    if s < 0:
        return jnp.concatenate([jnp.zeros((-s, C), v.dtype), v[:PIX + s]], axis=0)
    return v


def _moe_kernel(x_ref, g_ref, wt_ref, bc_ref, wp_ref, bp_ref,
                y1_ref, y2_ref, y3_ref, y4_ref, loss_ref, usage_ref):
    i = pl.program_id(0)
    xt = x_ref[0]

    ones_pix = jnp.ones((PIX, 1), F32)
    x0t = jnp.dot(xt, ones_pix, preferred_element_type=F32) * (1.0 / PIX)
    logits = jax.lax.dot_general(x0t, g_ref[...], (((0,), (0,)), ((), ())),
                                 preferred_element_type=F32)

    iota = jax.lax.broadcasted_iota(jnp.int32, (1, NUM_EXPERTS), 1)
    iota_c = jax.lax.broadcasted_iota(jnp.int32, (NUM_EXPERTS, 1), 0)
    coeff_rows = []
    coeff_cols = []
    prob_rows = []
    for g in range(NUM_GATES):
        lg = logits[:, g * NUM_EXPERTS:(g + 1) * NUM_EXPERTS]
        lg = lg - jnp.max(lg, axis=1, keepdims=True)
        el = jnp.exp(lg)
        p = el / jnp.sum(el, axis=1, keepdims=True)
        prob_rows.append(p)
        m0 = jnp.max(p, axis=1, keepdims=True)
        i0 = jnp.min(jnp.where(p == m0, iota, NUM_EXPERTS), axis=1, keepdims=True)
        pm = jnp.where(iota == i0, -jnp.inf, p)
        m1 = jnp.max(pm, axis=1, keepdims=True)
        i1 = jnp.min(jnp.where(pm == m1, iota, NUM_EXPERTS), axis=1, keepdims=True)
        t = jnp.exp(m1 - m0)
        w0 = 1.0 / (1.0 + t)
        w1 = 1.0 - w0
        coeff_rows.append(w0 * (iota == i0).astype(F32)
                          + w1 * (iota == i1).astype(F32))
        coeff_cols.append(w0 * (iota_c == i0).astype(F32)
                          + w1 * (iota_c == i1).astype(F32))
    probs = jnp.concatenate(prob_rows, axis=0)

    @pl.when(i == 0)
    def _():
        usage_ref[...] = probs

    @pl.when(i > 0)
    def _():
        usage_ref[...] += probs

    xb = jnp.transpose(xt.astype(BF16))
    pcol = jax.lax.broadcasted_iota(jnp.int32, (PIX, 1), 0) & (W - 1)
    taps = []
    for t in range(9):
        oy, ox = t // 3 - 1, t % 3 - 1
        sx = _shift_rows(xb, oy * W + ox)
        if ox == 1:
            sx = jnp.where(pcol == W - 1, BF16(0), sx)
        elif ox == -1:
            sx = jnp.where(pcol == 0, BF16(0), sx)
        taps.append(sx)
    x9 = jnp.concatenate(taps, axis=1)
    u = jnp.dot(x9, wt_ref[...], preferred_element_type=F32)
    ub = (u + bc_ref[...]).astype(BF16)

    r8 = jax.lax.broadcasted_iota(jnp.int32, (EALL, NUM_EXPERTS), 0) >> 7
    c8 = jax.lax.broadcasted_iota(jnp.int32, (EALL, NUM_EXPERTS), 1)
    sel = (r8 == c8).astype(F32)
    sq = jnp.dot(ub * ub, sel.astype(BF16), preferred_element_type=F32)
    f = sq / ((1.0 + sq) * (jnp.sqrt(sq) + 1e-8))
    fb = f.astype(BF16)
    s_all = jnp.concatenate(
        [ub[:, e * C:(e + 1) * C] * fb[:, e:e + 1] for e in range(NUM_EXPERTS)],
        axis=1)

    wp = wp_ref[...]
    y_refs = (y1_ref, y2_ref, y3_ref, y4_ref)
    for g in range(NUM_GATES):
        c_col = jnp.dot(sel, coeff_cols[g], preferred_element_type=F32)
        wpg = wp * c_col.astype(BF16)
        yg = jnp.dot(s_all, wpg, preferred_element_type=F32)
        bias = jnp.dot(coeff_rows[g], bp_ref[...], preferred_element_type=F32)
        yg = yg + bias
        y_refs[g][...] = jnp.transpose(yg)[None]

    @pl.when(i == B - 1)
    def _():
        usage = usage_ref[...]
        mean = jnp.mean(usage, axis=1, keepdims=True)
        var = jnp.sum((usage - mean) ** 2, axis=1, keepdims=True) / (NUM_EXPERTS - 1)
        cv = var / (mean * mean + 1e-10)
        total = jnp.sum(cv, axis=0, keepdims=True)
        loss_ref[...] = jnp.broadcast_to(total, (1, NUM_EXPERTS))


def kernel(x, gate1, gate2, gate3, gate4, Wc, bc, Wp, bp):
    xr = x.reshape(B, C, PIX)
    gcat = jnp.concatenate([gate1, gate2, gate3, gate4], axis=1)
    wt = jnp.transpose(Wc.astype(BF16), (3, 4, 2, 0, 1)).reshape(K9, EALL)
    bc_all = bc.reshape(1, EALL)
    wps = jnp.transpose(Wp[:, :, :, 0, 0].astype(BF16), (0, 2, 1)).reshape(EALL, C)

    grid = (B,)
    outs = pl.pallas_call(
        _moe_kernel,
        grid=grid,
        in_specs=[
            pl.BlockSpec((1, C, PIX), lambda i: (i, 0, 0)),
            pl.BlockSpec((C, NUM_GATES * NUM_EXPERTS), lambda i: (0, 0)),
            pl.BlockSpec((K9, EALL), lambda i: (0, 0)),
            pl.BlockSpec((1, EALL), lambda i: (0, 0)),
            pl.BlockSpec((EALL, C), lambda i: (0, 0)),
            pl.BlockSpec((NUM_EXPERTS, C), lambda i: (0, 0)),
        ],
        out_specs=[
            pl.BlockSpec((1, C, PIX), lambda i: (i, 0, 0)),
            pl.BlockSpec((1, C, PIX), lambda i: (i, 0, 0)),
            pl.BlockSpec((1, C, PIX), lambda i: (i, 0, 0)),
            pl.BlockSpec((1, C, PIX), lambda i: (i, 0, 0)),
            pl.BlockSpec((1, NUM_EXPERTS), lambda i: (0, 0)),
        ],
        out_shape=[
            jax.ShapeDtypeStruct((B, C, PIX), F32),
            jax.ShapeDtypeStruct((B, C, PIX), F32),
            jax.ShapeDtypeStruct((B, C, PIX), F32),
            jax.ShapeDtypeStruct((B, C, PIX), F32),
            jax.ShapeDtypeStruct((1, NUM_EXPERTS), F32),
        ],
        scratch_shapes=[pltpu.VMEM((NUM_GATES, NUM_EXPERTS), F32)],
        compiler_params=pltpu.CompilerParams(
            dimension_semantics=("arbitrary",)),
    )(xr, gcat, wt, bc_all, wps, bp)

    ys = [o.reshape(B, C, H, W) for o in outs[:4]]
    l = outs[4][0, 0].reshape(())
    return (ys[0], ys[1], ys[2], ys[3], l)
